# baseline (speedup 1.0000x reference)
import jax
import jax.numpy as jnp
from jax import lax
from jax.experimental import pallas as pl
from jax.experimental.pallas import tpu as pltpu
from jax.experimental.pallas import tpu_sc as plsc

N = 4096
B = 16
M = 512
C2 = 32
K = 3
BM = B * M

NC = 2
NS = 16
L = 16
NW = NC * NS
QPW = N // NW
GRP = QPW // L

_BIG = 3.0e38


def _nr_sqrt(x):
    xb = plsc.bitcast(x, jnp.int32)
    y = plsc.bitcast(jnp.int32(0x5F3759DF) - (xb >> 1), jnp.float32)
    for _ in range(3):
        y = y * (jnp.float32(1.5) - jnp.float32(0.5) * x * y * y)
    return x * y


def _body(unknown_hbm, known_hbm, bi_hbm, feats_hbm, out_hbm,
          kt_v, q_v, bi_v, w_loc, gi_loc, rows_v, out_loc, sem):
    c = lax.axis_index("c")
    s = lax.axis_index("s")
    wid = c * NS + s
    qb = wid * QPW

    cps = []
    for d in range(3):
        cps.append(pltpu.async_copy(unknown_hbm.at[pl.ds(d * N + qb, QPW)],
                                    q_v.at[d], sem))
        cps.append(pltpu.async_copy(known_hbm.at[pl.ds(d * BM, BM)],
                                    kt_v.at[d], sem))
    cps.append(pltpu.async_copy(bi_hbm.at[pl.ds(qb, QPW)], bi_v, sem))
    for cp in cps:
        cp.wait()

    d0 = jnp.full((L,), 0, jnp.int32)
    d1 = jnp.full((L,), 1, jnp.int32)
    d2c = jnp.full((L,), 2, jnp.int32)
    zf = jnp.full((L,), 0.0, jnp.float32)
    zi = jnp.full((L,), 0, jnp.int32)

    for g in range(GRP):
        off = g * L
        qx = q_v[0, pl.ds(off, L)]
        qy = q_v[1, pl.ds(off, L)]
        qz = q_v[2, pl.ds(off, L)]
        bi = bi_v[pl.ds(off, L)]
        iv0 = bi * jnp.int32(M)

        def step(j, carry):
            iv, m0, m1, m2, i0, i1, i2 = carry
            px = plsc.load_gather(kt_v, [d0, iv])
            py = plsc.load_gather(kt_v, [d1, iv])
            pz = plsc.load_gather(kt_v, [d2c, iv])
            dx = qx - px
            dy = qy - py
            dz = qz - pz
            dd = dx * dx + dy * dy + dz * dz
            cc = dd < m2
            m2n = jnp.where(cc, dd, m2)
            i2n = jnp.where(cc, iv, i2)
            cb = m2n < m1
            m1b = jnp.where(cb, m2n, m1)
            m2b = jnp.where(cb, m1, m2n)
            i1b = jnp.where(cb, i2n, i1)
            i2b = jnp.where(cb, i1, i2n)
            ca = m1b < m0
            m0a = jnp.where(ca, m1b, m0)
            m1a = jnp.where(ca, m0, m1b)
            i0a = jnp.where(ca, i1b, i0)
            i1a = jnp.where(ca, i0, i1b)
            return (iv + jnp.int32(1), m0a, m1a, m2b, i0a, i1a, i2b)

        iv, m0, m1, m2, i0, i1, i2 = lax.fori_loop(
            0, M, step,
            (iv0, zf + jnp.float32(_BIG), zf + jnp.float32(_BIG),
             zf + jnp.float32(_BIG), zi, zi, zi),
            unroll=8)

        ws = []
        for mk in (m0, m1, m2):
            dk = _nr_sqrt(jnp.maximum(mk, jnp.float32(1e-30)))
            ws.append(jnp.float32(1.0) / (dk + jnp.float32(1e-8)))
        tot = ws[0] + ws[1] + ws[2]
        for k, (wk, ik) in enumerate(zip(ws, (i0, i1, i2))):
            w_loc[k, pl.ds(off, L)] = wk / tot
            gi_loc[k, pl.ds(off, L)] = ik

    cps = []
    for k in range(K):
        cps.append(pltpu.async_copy(feats_hbm.at[gi_loc.at[k]],
                                    rows_v.at[k], sem))
    for cp in cps:
        cp.wait()

    qi = lax.iota(jnp.int32, L)
    k0 = jnp.full((L,), 0, jnp.int32)
    k1 = jnp.full((L,), 1, jnp.int32)
    k2 = jnp.full((L,), 2, jnp.int32)
    for g in range(GRP):
        off = g * L
        qv = qi + jnp.int32(off)
        w0 = w_loc[0, pl.ds(off, L)]
        w1 = w_loc[1, pl.ds(off, L)]
        w2 = w_loc[2, pl.ds(off, L)]
        for ch in range(C2):
            cs = jnp.full((L,), ch, jnp.int32)
            f0 = plsc.load_gather(rows_v, [k0, qv, cs])
            f1 = plsc.load_gather(rows_v, [k1, qv, cs])
            f2 = plsc.load_gather(rows_v, [k2, qv, cs])
            out_loc[ch, pl.ds(off, L)] = w0 * f0 + w1 * f1 + w2 * f2

    pltpu.sync_copy(out_loc, out_hbm.at[:, pl.ds(qb, QPW)])


@jax.jit
def _sc_call(unknown_f, known_f, batch_inds, feats_t):
    mesh = plsc.VectorSubcoreMesh(core_axis_name="c", subcore_axis_name="s")
    f = pl.kernel(
        _body,
        out_type=jax.ShapeDtypeStruct((C2, N), jnp.float32),
        mesh=mesh,
        compiler_params=pltpu.CompilerParams(use_tc_tiling_on_sc=False,
                                             needs_layout_passes=False),
        scratch_types=[
            pltpu.VMEM((3, BM), jnp.float32),
            pltpu.VMEM((3, QPW), jnp.float32),
            pltpu.VMEM((QPW,), jnp.int32),
            pltpu.VMEM((K, QPW), jnp.float32),
            pltpu.VMEM((K, QPW), jnp.int32),
            pltpu.VMEM((K, QPW, C2), jnp.float32),
            pltpu.VMEM((C2, QPW), jnp.float32),
            pltpu.SemaphoreType.DMA,
        ],
    )
    return f(unknown_f, known_f, batch_inds, feats_t)


def kernel(unknown, known, batch_inds, known_feats):
    unknown_f = unknown.T.reshape(-1)
    known_f = jnp.transpose(known, (2, 0, 1)).reshape(-1)
    bi = batch_inds.astype(jnp.int32)
    feats_t = jnp.transpose(known_feats, (0, 2, 1)).reshape(BM, C2)
    out = _sc_call(unknown_f, known_f, bi, feats_t)
    return out.T.reshape(N, C2, 1)


# speedup vs baseline: 2.4201x; 2.4201x over previous
import jax
import jax.numpy as jnp
from jax import lax
from jax.experimental import pallas as pl
from jax.experimental.pallas import tpu as pltpu
from jax.experimental.pallas import tpu_sc as plsc

N = 4096
B = 16
M = 512
C2 = 32
K = 3
BM = B * M

NC = 2
NS = 16
L = 16
NW = NC * NS
QPW = N // NW
GRP = QPW // L

_BIG = 3.0e38


def _nr_sqrt(x):
    xb = plsc.bitcast(x, jnp.int32)
    y = plsc.bitcast(jnp.int32(0x5F3759DF) - (xb >> 1), jnp.float32)
    for _ in range(3):
        y = y * (jnp.float32(1.5) - jnp.float32(0.5) * x * y * y)
    return x * y


def _body(unknown_hbm, known_hbm, bi_hbm, feats_hbm, out_hbm,
          kt_v, q_v, bi_v, w_loc, gi_loc, rows_v, out_loc, sem):
    c = lax.axis_index("c")
    s = lax.axis_index("s")
    wid = c * NS + s
    qb = wid * QPW

    cps = []
    for d in range(3):
        cps.append(pltpu.async_copy(unknown_hbm.at[pl.ds(d * N + qb, QPW)],
                                    q_v.at[d], sem))
        cps.append(pltpu.async_copy(known_hbm.at[pl.ds(d * BM, BM)],
                                    kt_v.at[d], sem))
    cps.append(pltpu.async_copy(bi_hbm.at[pl.ds(qb, QPW)], bi_v, sem))
    for cp in cps:
        cp.wait()

    d0 = jnp.full((L,), 0, jnp.int32)
    d1 = jnp.full((L,), 1, jnp.int32)
    d2c = jnp.full((L,), 2, jnp.int32)
    zf = jnp.full((L,), 0.0, jnp.float32)
    zi = jnp.full((L,), 0, jnp.int32)

    for g in range(GRP):
        off = g * L
        qx = q_v[0, pl.ds(off, L)]
        qy = q_v[1, pl.ds(off, L)]
        qz = q_v[2, pl.ds(off, L)]
        bi = bi_v[pl.ds(off, L)]

        def step(j, carry):
            iv, m0, m1, m2, i0, i1, i2 = carry
            px = plsc.load_gather(kt_v, [d0, iv])
            py = plsc.load_gather(kt_v, [d1, iv])
            pz = plsc.load_gather(kt_v, [d2c, iv])
            dx = qx - px
            dy = qy - py
            dz = qz - pz
            dd = dx * dx + dy * dy + dz * dz
            cc = dd < m2
            m2n = jnp.where(cc, dd, m2)
            i2n = jnp.where(cc, iv, i2)
            cb = m2n < m1
            m1b = jnp.where(cb, m2n, m1)
            m2b = jnp.where(cb, m1, m2n)
            i1b = jnp.where(cb, i2n, i1)
            i2b = jnp.where(cb, i1, i2n)
            ca = m1b < m0
            m0a = jnp.where(ca, m1b, m0)
            m1a = jnp.where(ca, m0, m1b)
            i0a = jnp.where(ca, i1b, i0)
            i1a = jnp.where(ca, i0, i1b)
            return (iv + jnp.int32(L), m0a, m1a, m2b, i0a, i1a, i2b)

        iv, m0, m1, m2, i0, i1, i2 = lax.fori_loop(
            0, M, step,
            (bi, zf + jnp.float32(_BIG), zf + jnp.float32(_BIG),
             zf + jnp.float32(_BIG), zi, zi, zi),
            unroll=8)

        ws = []
        for mk in (m0, m1, m2):
            dk = _nr_sqrt(jnp.maximum(mk, jnp.float32(1e-30)))
            ws.append(jnp.float32(1.0) / (dk + jnp.float32(1e-8)))
        tot = ws[0] + ws[1] + ws[2]
        for k, (wk, ik) in enumerate(zip(ws, (i0, i1, i2))):
            w_loc[k, pl.ds(off, L)] = wk / tot
            gk = ((ik & jnp.int32(B - 1)) << 9) | (ik >> 4)
            gi_loc[k, pl.ds(off, L)] = gk

    cps = []
    for k in range(K):
        cps.append(pltpu.async_copy(feats_hbm.at[gi_loc.at[k]],
                                    rows_v.at[k], sem))
    for cp in cps:
        cp.wait()

    for q in range(QPW):
        qs = jnp.full((L,), q, jnp.int32)
        w0 = plsc.load_gather(w_loc, [d0, qs])
        w1 = plsc.load_gather(w_loc, [d1, qs])
        w2 = plsc.load_gather(w_loc, [d2c, qs])
        for h in range(C2 // L):
            f0 = rows_v[0, q, pl.ds(h * L, L)]
            f1 = rows_v[1, q, pl.ds(h * L, L)]
            f2 = rows_v[2, q, pl.ds(h * L, L)]
            out_loc[pl.ds(q * C2 + h * L, L)] = w0 * f0 + w1 * f1 + w2 * f2

    pltpu.sync_copy(out_loc, out_hbm.at[pl.ds(qb * C2, QPW * C2)])


@jax.jit
def _sc_call(unknown_f, known_f, batch_inds, feats_t):
    mesh = plsc.VectorSubcoreMesh(core_axis_name="c", subcore_axis_name="s")
    f = pl.kernel(
        _body,
        out_type=jax.ShapeDtypeStruct((N * C2,), jnp.float32),
        mesh=mesh,
        compiler_params=pltpu.CompilerParams(use_tc_tiling_on_sc=False,
                                             needs_layout_passes=False),
        scratch_types=[
            pltpu.VMEM((3, BM), jnp.float32),
            pltpu.VMEM((3, QPW), jnp.float32),
            pltpu.VMEM((QPW,), jnp.int32),
            pltpu.VMEM((K, QPW), jnp.float32),
            pltpu.VMEM((K, QPW), jnp.int32),
            pltpu.VMEM((K, QPW, C2), jnp.float32),
            pltpu.VMEM((QPW * C2,), jnp.float32),
            pltpu.SemaphoreType.DMA,
        ],
    )
    return f(unknown_f, known_f, batch_inds, feats_t)


def kernel(unknown, known, batch_inds, known_feats):
    unknown_f = unknown.T.reshape(-1)
    known_f = jnp.transpose(known, (2, 1, 0)).reshape(-1)
    bi = batch_inds.astype(jnp.int32)
    feats_t = jnp.transpose(known_feats, (0, 2, 1)).reshape(BM, C2)
    out = _sc_call(unknown_f, known_f, bi, feats_t)
    return out.reshape(N, C2, 1)
